# Initial kernel scaffold; baseline (speedup 1.0000x reference)
import jax
import jax.numpy as jnp
from jax.experimental import pallas as pl


def kernel(x_user, x_item, edge_index_user_item, edge_index_item_user, params):
    raise NotImplementedError("write your pallas kernel here")


# speedup vs baseline: 55.9268x; 55.9268x over previous
import functools
import math

import jax
import jax.numpy as jnp
from jax import lax
from jax.experimental import pallas as pl
from jax.experimental.pallas import tpu as pltpu
from jax.experimental.pallas import tpu_sc as plsc

HEADS = 2
D = 16
HID = HEADS * D
EPS = 1e-16


_TC_R = 1000


def _front1_body(x_ref, wp_ref, bp_ref, wq_ref, bq_ref, wk_ref, bk_ref,
                 wv_ref, bv_ref, h_ref, qt_ref, kt_ref, vt_ref):
  x = x_ref[...]
  h = jnp.maximum(
      jnp.dot(x, wp_ref[...], preferred_element_type=jnp.float32)
      + bp_ref[...], 0.0)
  q = jnp.dot(h, wq_ref[...], preferred_element_type=jnp.float32) + bq_ref[...]
  k = jnp.dot(h, wk_ref[...], preferred_element_type=jnp.float32) + bk_ref[...]
  v = jnp.dot(h, wv_ref[...], preferred_element_type=jnp.float32) + bv_ref[...]
  h_ref[...] = h
  qt_ref[0] = q[:, :D]
  qt_ref[1] = q[:, D:]
  kt_ref[0] = k[:, :D]
  kt_ref[1] = k[:, D:]
  vt_ref[0] = v[:, :D]
  vt_ref[1] = v[:, D:]


def _front2_body(h_ref, wq_ref, bq_ref, wk_ref, bk_ref, wv_ref, bv_ref,
                 qt_ref, kt_ref, vt_ref):
  h = h_ref[...]
  q = jnp.dot(h, wq_ref[...], preferred_element_type=jnp.float32) + bq_ref[...]
  k = jnp.dot(h, wk_ref[...], preferred_element_type=jnp.float32) + bk_ref[...]
  v = jnp.dot(h, wv_ref[...], preferred_element_type=jnp.float32) + bv_ref[...]
  qt_ref[0] = q[:, :D]
  qt_ref[1] = q[:, D:]
  kt_ref[0] = k[:, :D]
  kt_ref[1] = k[:, D:]
  vt_ref[0] = v[:, :D]
  vt_ref[1] = v[:, D:]


def _back_body(nd_ref, h_ref, wo_ref, bo_ref, skip_ref, out_ref, *, relu):
  num0 = nd_ref[0, :, :D]
  den0 = nd_ref[0, :, D:D + 1]
  num1 = nd_ref[1, :, :D]
  den1 = nd_ref[1, :, D:D + 1]
  agg = jnp.concatenate([num0 / (den0 + EPS), num1 / (den1 + EPS)], axis=-1)
  o = 0.5 * agg * (1.0 + lax.erf(agg * (1.0 / math.sqrt(2.0))))
  o = jnp.dot(o, wo_ref[...], preferred_element_type=jnp.float32) + bo_ref[...]
  beta = jax.nn.sigmoid(skip_ref[0, 0])
  out = beta * o + (1.0 - beta) * h_ref[...]
  if relu:
    out = jnp.maximum(out, 0.0)
  out_ref[...] = out


def _w_spec(shape):
  return pl.BlockSpec(shape, lambda i: (0,) * len(shape))


def _front1(x, wp, bp, wq, bq, wk, bk, wv, bv):
  n, in_dim = x.shape
  r = _TC_R
  grid = (n // r,)
  f32 = jnp.float32
  return pl.pallas_call(
      _front1_body,
      grid=grid,
      in_specs=[
          pl.BlockSpec((r, in_dim), lambda i: (i, 0)),
          _w_spec((in_dim, HID)), _w_spec((1, HID)),
          _w_spec((HID, HID)), _w_spec((1, HID)),
          _w_spec((HID, HID)), _w_spec((1, HID)),
          _w_spec((HID, HID)), _w_spec((1, HID)),
      ],
      out_specs=[
          pl.BlockSpec((r, HID), lambda i: (i, 0)),
          pl.BlockSpec((HEADS, r, D), lambda i: (0, i, 0)),
          pl.BlockSpec((HEADS, r, D), lambda i: (0, i, 0)),
          pl.BlockSpec((HEADS, r, D), lambda i: (0, i, 0)),
      ],
      out_shape=[
          jax.ShapeDtypeStruct((n, HID), f32),
          jax.ShapeDtypeStruct((HEADS, n, D), f32),
          jax.ShapeDtypeStruct((HEADS, n, D), f32),
          jax.ShapeDtypeStruct((HEADS, n, D), f32),
      ],
  )(x, wp, bp, wq, bq, wk, bk, wv, bv)


def _front2(h, wq, bq, wk, bk, wv, bv):
  n = h.shape[0]
  r = _TC_R
  f32 = jnp.float32
  return pl.pallas_call(
      _front2_body,
      grid=(n // r,),
      in_specs=[
          pl.BlockSpec((r, HID), lambda i: (i, 0)),
          _w_spec((HID, HID)), _w_spec((1, HID)),
          _w_spec((HID, HID)), _w_spec((1, HID)),
          _w_spec((HID, HID)), _w_spec((1, HID)),
      ],
      out_specs=[
          pl.BlockSpec((HEADS, r, D), lambda i: (0, i, 0)),
          pl.BlockSpec((HEADS, r, D), lambda i: (0, i, 0)),
          pl.BlockSpec((HEADS, r, D), lambda i: (0, i, 0)),
      ],
      out_shape=[
          jax.ShapeDtypeStruct((HEADS, n, D), f32),
          jax.ShapeDtypeStruct((HEADS, n, D), f32),
          jax.ShapeDtypeStruct((HEADS, n, D), f32),
      ],
  )(h, wq, bq, wk, bk, wv, bv)


def _back(nd, h, wo, bo, skip, relu):
  n = h.shape[0]
  r = _TC_R
  return pl.pallas_call(
      functools.partial(_back_body, relu=relu),
      grid=(n // r,),
      in_specs=[
          pl.BlockSpec((HEADS, r, 24), lambda i: (0, i, 0)),
          pl.BlockSpec((r, HID), lambda i: (i, 0)),
          _w_spec((HID, HID)), _w_spec((1, HID)),
          _w_spec((1, 1)),
      ],
      out_specs=pl.BlockSpec((r, HID), lambda i: (i, 0)),
      out_shape=jax.ShapeDtypeStruct((n, HID), jnp.float32),
  )(nd, h, wo, bo, skip)


_C = 640
_ZR = 125


@functools.cache
def _make_sc_et(n, e):
  assert e % _C == 0
  nchunks = e // _C
  groups = _C // 16
  rpt = ((n // 16 + 7) // 8) * 8
  n_pad = 16 * rpt
  zr = _ZR if rpt % _ZR == 0 else 8
  f32 = jnp.float32
  i32 = jnp.int32
  mesh = plsc.VectorSubcoreMesh(core_axis_name="c", subcore_axis_name="s")

  @functools.partial(
      pl.kernel,
      mesh=mesh,
      out_type=(jax.ShapeDtypeStruct((HEADS, n_pad, 24), f32),
                jax.ShapeDtypeStruct((HEADS, e), f32)),
      compiler_params=pltpu.CompilerParams(
          use_tc_tiling_on_sc=False, needs_layout_passes=False),
      scratch_types=[
          pltpu.VMEM((_C,), i32),
          pltpu.VMEM((_C,), i32),
          pltpu.VMEM((_C, 16), f32),
          pltpu.VMEM((_C, 16), f32),
          pltpu.VMEM((_C,), f32),
          pltpu.VMEM((272,), f32),
          pltpu.VMEM((16,), f32),
          pltpu.VMEM((_C, 24), f32),
          pltpu.VMEM((zr, 24), f32),
          pltpu.VMEM((16, 16), f32),
          pltpu.VMEM_SHARED((n_pad, 24), f32),
          pltpu.VMEM_SHARED((16, 16), f32),
          pltpu.SemaphoreType.DMA,
          pltpu.SemaphoreType.DMA,
      ],
  )
  def sc_et(src_hbm, dst_hbm, ktab, qtab, vtab, nd_out, alpha_out,
            sidx, didx, gbuf, qbuf, abuf, scr, mbuf, msgbuf, zbuf, tred,
            acc, tmax, sem1, sem2):
    c = lax.axis_index("c")
    s = lax.axis_index("s")
    iota = lax.iota(i32, 16)
    idx_pitch17 = iota * 17 + 15
    idx_15 = (iota * 0) + 15
    zero16 = jnp.zeros((16,), f32)

    def zrow(r, _):
      zbuf[r, pl.ds(8, 16)] = zero16
      zbuf[r, pl.ds(0, 16)] = zero16
      return _
    lax.fori_loop(0, zr, zrow, None)

    def zcopy(z, _):
      pltpu.sync_copy(zbuf, acc.at[pl.ds(s * rpt + z * zr, zr), :])
      return _
    lax.fori_loop(0, rpt // zr, zcopy, None)
    plsc.subcore_barrier()

    nct = (jnp.int32(nchunks) - s + 15) // 16
    mbuf[...] = jnp.full((16,), -1e30, f32)

    def chunk_a(j, _):
      base = (s + 16 * j) * _C
      pltpu.sync_copy(src_hbm.at[pl.ds(base, _C)], sidx)
      pltpu.sync_copy(dst_hbm.at[pl.ds(base, _C)], didx)
      cp1 = pltpu.async_copy(ktab.at[c].at[sidx], gbuf, sem1)
      cp2 = pltpu.async_copy(qtab.at[c].at[didx], qbuf, sem2)
      cp1.wait()
      cp2.wait()

      def group_a(g, _):
        for ee in range(16):
          i = g * 16 + ee
          prod = gbuf[i] * qbuf[i]
          scr[pl.ds(17 * ee, 16)] = plsc.cumsum(prod)
        avec = plsc.load_gather(scr, [idx_pitch17])
        abuf[pl.ds(g * 16, 16)] = avec
        mbuf[...] = jnp.maximum(mbuf[...], avec)
        return _
      lax.fori_loop(0, groups, group_a, None)
      pltpu.sync_copy(abuf, alpha_out.at[c, pl.ds(base, _C)])
      return _
    lax.fori_loop(0, nct, chunk_a, None)

    pltpu.sync_copy(mbuf, tmax.at[s])
    plsc.subcore_barrier()
    pltpu.sync_copy(tmax, tred)
    m = jnp.full((16,), -1e30, f32)
    for r in range(16):
      m = jnp.maximum(m, tred[r])
    scr[pl.ds(0, 16)] = plsc.cummax(m)
    svec = plsc.load_gather(scr, [idx_15])

    def chunk_b(j, _):
      base = (s + 16 * j) * _C
      pltpu.sync_copy(src_hbm.at[pl.ds(base, _C)], sidx)
      pltpu.sync_copy(dst_hbm.at[pl.ds(base, _C)], didx)
      cp1 = pltpu.async_copy(vtab.at[c].at[sidx], gbuf, sem1)
      pltpu.sync_copy(alpha_out.at[c, pl.ds(base, _C)], abuf)
      cp1.wait()

      def group_b(g, _):
        av = abuf[pl.ds(g * 16, 16)]
        w = jnp.exp(av - svec)
        scr[pl.ds(0, 16)] = w
        for ee in range(16):
          i = g * 16 + ee
          sp = plsc.load_gather(scr, [(iota * 0) + ee])
          msgbuf[i, pl.ds(8, 16)] = jnp.where(iota == 8, sp, 0.0)
          msgbuf[i, pl.ds(0, 16)] = sp * gbuf[i]
        return _
      lax.fori_loop(0, groups, group_b, None)
      pltpu.sync_copy(msgbuf, acc.at[didx], add=True)
      return _
    lax.fori_loop(0, nct, chunk_b, None)

    plsc.subcore_barrier()
    pltpu.sync_copy(acc.at[pl.ds(s * rpt, rpt), :],
                    nd_out.at[c, pl.ds(s * rpt, rpt), :])

  return sc_et


def _eff_weights(cp, nt, rel):
  scale = 1.0 / math.sqrt(D)
  a = cp["a_" + rel]
  m = cp["m_" + rel]
  p = cp["p_" + rel]
  za = jnp.zeros((HID, HID), jnp.float32)
  za = za.at[:D, :D].set(a[0] * (p[0] * scale))
  za = za.at[D:, D:].set(a[1] * (p[1] * scale))
  zm = jnp.zeros((HID, HID), jnp.float32)
  zm = zm.at[:D, :D].set(m[0])
  zm = zm.at[D:, D:].set(m[1])
  wk = cp["Wk_" + nt] @ za
  bk = (cp["bk_" + nt] @ za).reshape(1, HID)
  wv = cp["Wv_" + nt] @ zm
  bv = (cp["bv_" + nt] @ zm).reshape(1, HID)
  wq = cp["Wq_" + nt]
  bq = cp["bq_" + nt].reshape(1, HID)
  return wq, bq, wk, bk, wv, bv


def kernel(x_user, x_item, edge_index_user_item, edge_index_item_user, params):
  n = x_user.shape[0]
  e = edge_index_user_item.shape[1]
  sc_et = _make_sc_et(n, e)

  src_ui = edge_index_user_item[0]
  dst_ui = edge_index_user_item[1]
  src_iu = edge_index_item_user[0]
  dst_iu = edge_index_item_user[1]

  c1, c2 = params["c1"], params["c2"]
  eff1_u = _eff_weights(c1, "user", "to")
  eff1_i = _eff_weights(c1, "item", "rev")
  eff2_u = _eff_weights(c2, "user", "to")
  eff2_i = _eff_weights(c2, "item", "rev")

  h_u, qt_u, kt_u, vt_u = _front1(
      x_user, params["Wp_user"], params["bp_user"].reshape(1, HID), *eff1_u)
  h_i, qt_i, kt_i, vt_i = _front1(
      x_item, params["Wp_item"], params["bp_item"].reshape(1, HID), *eff1_i)

  nd_to, _ = sc_et(src_ui, dst_ui, kt_u, qt_i, vt_u)
  nd_rev, _ = sc_et(src_iu, dst_iu, kt_i, qt_u, vt_i)

  h2_u = _back(nd_rev, h_u, c1["Wo_user"], c1["bo_user"].reshape(1, HID),
               c1["skip_user"].reshape(1, 1), relu=True)
  h2_i = _back(nd_to, h_i, c1["Wo_item"], c1["bo_item"].reshape(1, HID),
               c1["skip_item"].reshape(1, 1), relu=True)

  qt2_u, kt2_u, vt2_u = _front2(h2_u, *eff2_u)
  qt2_i, kt2_i, vt2_i = _front2(h2_i, *eff2_i)

  nd_to2, _ = sc_et(src_ui, dst_ui, kt2_u, qt2_i, vt2_u)
  nd_rev2, _ = sc_et(src_iu, dst_iu, kt2_i, qt2_u, vt2_i)

  out_u = _back(nd_rev2, h2_u, c2["Wo_user"], c2["bo_user"].reshape(1, HID),
                c2["skip_user"].reshape(1, 1), relu=False)
  out_i = _back(nd_to2, h2_i, c2["Wo_item"], c2["bo_item"].reshape(1, HID),
                c2["skip_item"].reshape(1, 1), relu=False)
  return out_u, out_i


# speedup vs baseline: 74.3170x; 1.3288x over previous
import functools
import math

import jax
import jax.numpy as jnp
from jax import lax
from jax.experimental import pallas as pl
from jax.experimental.pallas import tpu as pltpu
from jax.experimental.pallas import tpu_sc as plsc

HEADS = 2
D = 16
HID = HEADS * D
EPS = 1e-16


_TC_R = 1000


def _front1_body(x_ref, wp_ref, bp_ref, wq_ref, bq_ref, wk_ref, bk_ref,
                 wv_ref, bv_ref, h_ref, qt_ref, kt_ref, vt_ref):
  x = x_ref[...]
  h = jnp.maximum(
      jnp.dot(x, wp_ref[...], preferred_element_type=jnp.float32)
      + bp_ref[...], 0.0)
  q = jnp.dot(h, wq_ref[...], preferred_element_type=jnp.float32) + bq_ref[...]
  k = jnp.dot(h, wk_ref[...], preferred_element_type=jnp.float32) + bk_ref[...]
  v = jnp.dot(h, wv_ref[...], preferred_element_type=jnp.float32) + bv_ref[...]
  h_ref[...] = h
  qt_ref[0] = q[:, :D]
  qt_ref[1] = q[:, D:]
  kt_ref[0] = k[:, :D]
  kt_ref[1] = k[:, D:]
  vt_ref[0] = v[:, :D]
  vt_ref[1] = v[:, D:]


def _front2_body(h_ref, wq_ref, bq_ref, wk_ref, bk_ref, wv_ref, bv_ref,
                 qt_ref, kt_ref, vt_ref):
  h = h_ref[...]
  q = jnp.dot(h, wq_ref[...], preferred_element_type=jnp.float32) + bq_ref[...]
  k = jnp.dot(h, wk_ref[...], preferred_element_type=jnp.float32) + bk_ref[...]
  v = jnp.dot(h, wv_ref[...], preferred_element_type=jnp.float32) + bv_ref[...]
  qt_ref[0] = q[:, :D]
  qt_ref[1] = q[:, D:]
  kt_ref[0] = k[:, :D]
  kt_ref[1] = k[:, D:]
  vt_ref[0] = v[:, :D]
  vt_ref[1] = v[:, D:]


def _back_body(nd_ref, h_ref, wo_ref, bo_ref, skip_ref, out_ref, *, relu):
  num0 = nd_ref[0, :, :D]
  den0 = nd_ref[0, :, D:D + 1]
  num1 = nd_ref[1, :, :D]
  den1 = nd_ref[1, :, D:D + 1]
  agg = jnp.concatenate([num0 / (den0 + EPS), num1 / (den1 + EPS)], axis=-1)
  o = 0.5 * agg * (1.0 + lax.erf(agg * (1.0 / math.sqrt(2.0))))
  o = jnp.dot(o, wo_ref[...], preferred_element_type=jnp.float32) + bo_ref[...]
  beta = jax.nn.sigmoid(skip_ref[0, 0])
  out = beta * o + (1.0 - beta) * h_ref[...]
  if relu:
    out = jnp.maximum(out, 0.0)
  out_ref[...] = out


def _w_spec(shape):
  return pl.BlockSpec(shape, lambda i: (0,) * len(shape))


def _front1(x, wp, bp, wq, bq, wk, bk, wv, bv):
  n, in_dim = x.shape
  r = _TC_R
  grid = (n // r,)
  f32 = jnp.float32
  return pl.pallas_call(
      _front1_body,
      grid=grid,
      in_specs=[
          pl.BlockSpec((r, in_dim), lambda i: (i, 0)),
          _w_spec((in_dim, HID)), _w_spec((1, HID)),
          _w_spec((HID, HID)), _w_spec((1, HID)),
          _w_spec((HID, HID)), _w_spec((1, HID)),
          _w_spec((HID, HID)), _w_spec((1, HID)),
      ],
      out_specs=[
          pl.BlockSpec((r, HID), lambda i: (i, 0)),
          pl.BlockSpec((HEADS, r, D), lambda i: (0, i, 0)),
          pl.BlockSpec((HEADS, r, D), lambda i: (0, i, 0)),
          pl.BlockSpec((HEADS, r, D), lambda i: (0, i, 0)),
      ],
      out_shape=[
          jax.ShapeDtypeStruct((n, HID), f32),
          jax.ShapeDtypeStruct((HEADS, n, D), f32),
          jax.ShapeDtypeStruct((HEADS, n, D), f32),
          jax.ShapeDtypeStruct((HEADS, n, D), f32),
      ],
  )(x, wp, bp, wq, bq, wk, bk, wv, bv)


def _front2(h, wq, bq, wk, bk, wv, bv):
  n = h.shape[0]
  r = _TC_R
  f32 = jnp.float32
  return pl.pallas_call(
      _front2_body,
      grid=(n // r,),
      in_specs=[
          pl.BlockSpec((r, HID), lambda i: (i, 0)),
          _w_spec((HID, HID)), _w_spec((1, HID)),
          _w_spec((HID, HID)), _w_spec((1, HID)),
          _w_spec((HID, HID)), _w_spec((1, HID)),
      ],
      out_specs=[
          pl.BlockSpec((HEADS, r, D), lambda i: (0, i, 0)),
          pl.BlockSpec((HEADS, r, D), lambda i: (0, i, 0)),
          pl.BlockSpec((HEADS, r, D), lambda i: (0, i, 0)),
      ],
      out_shape=[
          jax.ShapeDtypeStruct((HEADS, n, D), f32),
          jax.ShapeDtypeStruct((HEADS, n, D), f32),
          jax.ShapeDtypeStruct((HEADS, n, D), f32),
      ],
  )(h, wq, bq, wk, bk, wv, bv)


def _back(nd, h, wo, bo, skip, relu):
  n = h.shape[0]
  r = _TC_R
  return pl.pallas_call(
      functools.partial(_back_body, relu=relu),
      grid=(n // r,),
      in_specs=[
          pl.BlockSpec((HEADS, r, 24), lambda i: (0, i, 0)),
          pl.BlockSpec((r, HID), lambda i: (i, 0)),
          _w_spec((HID, HID)), _w_spec((1, HID)),
          _w_spec((1, 1)),
      ],
      out_specs=pl.BlockSpec((r, HID), lambda i: (i, 0)),
      out_shape=jax.ShapeDtypeStruct((n, HID), jnp.float32),
  )(nd, h, wo, bo, skip)


_C = 640
_ZR = 125


@functools.cache
def _make_sc_et(n, e):
  assert e % _C == 0
  nchunks = e // _C
  groups = _C // 16
  rpt = ((n // 16 + 7) // 8) * 8
  n_pad = 16 * rpt
  zr = _ZR if rpt % _ZR == 0 else 8
  f32 = jnp.float32
  i32 = jnp.int32
  mesh = plsc.VectorSubcoreMesh(core_axis_name="c", subcore_axis_name="s")

  @functools.partial(
      pl.kernel,
      mesh=mesh,
      out_type=(jax.ShapeDtypeStruct((HEADS, n_pad, 24), f32),
                jax.ShapeDtypeStruct((HEADS, e), f32)),
      compiler_params=pltpu.CompilerParams(
          use_tc_tiling_on_sc=False, needs_layout_passes=False),
      scratch_types=[
          pltpu.VMEM((_C,), i32),
          pltpu.VMEM((_C,), i32),
          pltpu.VMEM((_C, 16), f32),
          pltpu.VMEM((_C, 16), f32),
          pltpu.VMEM((_C,), f32),
          pltpu.VMEM((272,), f32),
          pltpu.VMEM((16,), f32),
          pltpu.VMEM((_C, 24), f32),
          pltpu.VMEM((zr, 24), f32),
          pltpu.VMEM((16, 16), f32),
          pltpu.VMEM_SHARED((n_pad, 24), f32),
          pltpu.VMEM_SHARED((16, 16), f32),
          pltpu.SemaphoreType.DMA,
          pltpu.SemaphoreType.DMA,
      ],
  )
  def sc_et(src_hbm, dst_hbm, ktab, qtab, vtab, nd_out, alpha_out,
            sidx, didx, gbuf, qbuf, abuf, scr, mbuf, msgbuf, zbuf, tred,
            acc, tmax, sem1, sem2):
    c = lax.axis_index("c")
    s = lax.axis_index("s")
    iota = lax.iota(i32, 16)
    idx_pitch0 = iota * 17
    idx_15 = (iota * 0) + 15
    zero16 = jnp.zeros((16,), f32)

    def zrow(r, _):
      zbuf[r, pl.ds(8, 16)] = zero16
      zbuf[r, pl.ds(0, 16)] = zero16
      return _
    lax.fori_loop(0, zr, zrow, None)

    def zcopy(z, _):
      pltpu.sync_copy(zbuf, acc.at[pl.ds(s * rpt + z * zr, zr), :])
      return _
    lax.fori_loop(0, rpt // zr, zcopy, None)

    def zmrow(r, _):
      msgbuf[r, pl.ds(8, 16)] = zero16
      return _
    lax.fori_loop(0, _C, zmrow, None)
    plsc.subcore_barrier()

    nct = (jnp.int32(nchunks) - s + 15) // 16
    mbuf[...] = jnp.full((16,), -1e30, f32)

    def chunk_a(j, _):
      base = (s + 16 * j) * _C
      pltpu.sync_copy(src_hbm.at[pl.ds(base, _C)], sidx)
      pltpu.sync_copy(dst_hbm.at[pl.ds(base, _C)], didx)
      cp1 = pltpu.async_copy(ktab.at[c].at[sidx], gbuf, sem1)
      cp2 = pltpu.async_copy(qtab.at[c].at[didx], qbuf, sem2)
      cp1.wait()
      cp2.wait()

      def group_a(g, _):
        for ee in range(16):
          i = g * 16 + ee
          scr[pl.ds(17 * ee, 16)] = gbuf[i] * qbuf[i]
        avec = jnp.zeros((16,), f32)
        for d in range(16):
          avec = avec + plsc.load_gather(scr, [idx_pitch0 + d])
        abuf[pl.ds(g * 16, 16)] = avec
        mbuf[...] = jnp.maximum(mbuf[...], avec)
        return _
      lax.fori_loop(0, groups, group_a, None)
      pltpu.sync_copy(abuf, alpha_out.at[c, pl.ds(base, _C)])
      return _
    lax.fori_loop(0, nct, chunk_a, None)

    pltpu.sync_copy(mbuf, tmax.at[s])
    plsc.subcore_barrier()
    pltpu.sync_copy(tmax, tred)
    m = jnp.full((16,), -1e30, f32)
    for r in range(16):
      m = jnp.maximum(m, tred[r])
    scr[pl.ds(0, 16)] = plsc.cummax(m)
    svec = plsc.load_gather(scr, [idx_15])

    def chunk_b(j, _):
      base = (s + 16 * j) * _C
      pltpu.sync_copy(src_hbm.at[pl.ds(base, _C)], sidx)
      pltpu.sync_copy(dst_hbm.at[pl.ds(base, _C)], didx)
      cp1 = pltpu.async_copy(vtab.at[c].at[sidx], gbuf, sem1)
      pltpu.sync_copy(alpha_out.at[c, pl.ds(base, _C)], abuf)
      cp1.wait()

      def group_b(g, _):
        evec = iota + g * 16
        wvec = jnp.exp(abuf[pl.ds(g * 16, 16)] - svec)
        scr[pl.ds(0, 16)] = wvec
        plsc.store_scatter(msgbuf, [evec, (iota * 0) + 16], wvec)
        for ee in range(16):
          i = g * 16 + ee
          sp = plsc.load_gather(scr, [(iota * 0) + ee])
          msgbuf[i, pl.ds(0, 16)] = sp * gbuf[i]
        return _
      lax.fori_loop(0, groups, group_b, None)
      pltpu.sync_copy(msgbuf, acc.at[didx], add=True)
      return _
    lax.fori_loop(0, nct, chunk_b, None)

    plsc.subcore_barrier()
    pltpu.sync_copy(acc.at[pl.ds(s * rpt, rpt), :],
                    nd_out.at[c, pl.ds(s * rpt, rpt), :])

  return sc_et


def _eff_weights(cp, nt, rel):
  scale = 1.0 / math.sqrt(D)
  a = cp["a_" + rel]
  m = cp["m_" + rel]
  p = cp["p_" + rel]
  za = jnp.zeros((HID, HID), jnp.float32)
  za = za.at[:D, :D].set(a[0] * (p[0] * scale))
  za = za.at[D:, D:].set(a[1] * (p[1] * scale))
  zm = jnp.zeros((HID, HID), jnp.float32)
  zm = zm.at[:D, :D].set(m[0])
  zm = zm.at[D:, D:].set(m[1])
  wk = cp["Wk_" + nt] @ za
  bk = (cp["bk_" + nt] @ za).reshape(1, HID)
  wv = cp["Wv_" + nt] @ zm
  bv = (cp["bv_" + nt] @ zm).reshape(1, HID)
  wq = cp["Wq_" + nt]
  bq = cp["bq_" + nt].reshape(1, HID)
  return wq, bq, wk, bk, wv, bv


def kernel(x_user, x_item, edge_index_user_item, edge_index_item_user, params):
  n = x_user.shape[0]
  e = edge_index_user_item.shape[1]
  sc_et = _make_sc_et(n, e)

  src_ui = edge_index_user_item[0]
  dst_ui = edge_index_user_item[1]
  src_iu = edge_index_item_user[0]
  dst_iu = edge_index_item_user[1]

  c1, c2 = params["c1"], params["c2"]
  eff1_u = _eff_weights(c1, "user", "to")
  eff1_i = _eff_weights(c1, "item", "rev")
  eff2_u = _eff_weights(c2, "user", "to")
  eff2_i = _eff_weights(c2, "item", "rev")

  h_u, qt_u, kt_u, vt_u = _front1(
      x_user, params["Wp_user"], params["bp_user"].reshape(1, HID), *eff1_u)
  h_i, qt_i, kt_i, vt_i = _front1(
      x_item, params["Wp_item"], params["bp_item"].reshape(1, HID), *eff1_i)

  nd_to, _ = sc_et(src_ui, dst_ui, kt_u, qt_i, vt_u)
  nd_rev, _ = sc_et(src_iu, dst_iu, kt_i, qt_u, vt_i)

  h2_u = _back(nd_rev, h_u, c1["Wo_user"], c1["bo_user"].reshape(1, HID),
               c1["skip_user"].reshape(1, 1), relu=True)
  h2_i = _back(nd_to, h_i, c1["Wo_item"], c1["bo_item"].reshape(1, HID),
               c1["skip_item"].reshape(1, 1), relu=True)

  qt2_u, kt2_u, vt2_u = _front2(h2_u, *eff2_u)
  qt2_i, kt2_i, vt2_i = _front2(h2_i, *eff2_i)

  nd_to2, _ = sc_et(src_ui, dst_ui, kt2_u, qt2_i, vt2_u)
  nd_rev2, _ = sc_et(src_iu, dst_iu, kt2_i, qt2_u, vt2_i)

  out_u = _back(nd_rev2, h2_u, c2["Wo_user"], c2["bo_user"].reshape(1, HID),
                c2["skip_user"].reshape(1, 1), relu=False)
  out_i = _back(nd_to2, h2_i, c2["Wo_item"], c2["bo_item"].reshape(1, HID),
                c2["skip_item"].reshape(1, 1), relu=False)
  return out_u, out_i


# speedup vs baseline: 87.1428x; 1.1726x over previous
import functools
import math

import jax
import jax.numpy as jnp
from jax import lax
from jax.experimental import pallas as pl
from jax.experimental.pallas import tpu as pltpu
from jax.experimental.pallas import tpu_sc as plsc

HEADS = 2
D = 16
HID = HEADS * D
EPS = 1e-16


_TC_R = 1000


def _front1_body(x_ref, wp_ref, bp_ref, wq_ref, bq_ref, wk_ref, bk_ref,
                 wv_ref, bv_ref, h_ref, qt_ref, kt_ref, vt_ref):
  x = x_ref[...]
  h = jnp.maximum(
      jnp.dot(x, wp_ref[...], preferred_element_type=jnp.float32)
      + bp_ref[...], 0.0)
  q = jnp.dot(h, wq_ref[...], preferred_element_type=jnp.float32) + bq_ref[...]
  k = jnp.dot(h, wk_ref[...], preferred_element_type=jnp.float32) + bk_ref[...]
  v = jnp.dot(h, wv_ref[...], preferred_element_type=jnp.float32) + bv_ref[...]
  h_ref[...] = h
  qt_ref[0] = q[:, :D]
  qt_ref[1] = q[:, D:]
  kt_ref[0] = k[:, :D]
  kt_ref[1] = k[:, D:]
  vt_ref[0] = v[:, :D]
  vt_ref[1] = v[:, D:]


def _front2_body(h_ref, wq_ref, bq_ref, wk_ref, bk_ref, wv_ref, bv_ref,
                 qt_ref, kt_ref, vt_ref):
  h = h_ref[...]
  q = jnp.dot(h, wq_ref[...], preferred_element_type=jnp.float32) + bq_ref[...]
  k = jnp.dot(h, wk_ref[...], preferred_element_type=jnp.float32) + bk_ref[...]
  v = jnp.dot(h, wv_ref[...], preferred_element_type=jnp.float32) + bv_ref[...]
  qt_ref[0] = q[:, :D]
  qt_ref[1] = q[:, D:]
  kt_ref[0] = k[:, :D]
  kt_ref[1] = k[:, D:]
  vt_ref[0] = v[:, :D]
  vt_ref[1] = v[:, D:]


def _back_body(nd_ref, h_ref, wo_ref, bo_ref, skip_ref, out_ref, *, relu):
  num0 = nd_ref[0, :, :D]
  den0 = nd_ref[0, :, D:D + 1]
  num1 = nd_ref[1, :, :D]
  den1 = nd_ref[1, :, D:D + 1]
  agg = jnp.concatenate([num0 / (den0 + EPS), num1 / (den1 + EPS)], axis=-1)
  o = 0.5 * agg * (1.0 + lax.erf(agg * (1.0 / math.sqrt(2.0))))
  o = jnp.dot(o, wo_ref[...], preferred_element_type=jnp.float32) + bo_ref[...]
  beta = jax.nn.sigmoid(skip_ref[0, 0])
  out = beta * o + (1.0 - beta) * h_ref[...]
  if relu:
    out = jnp.maximum(out, 0.0)
  out_ref[...] = out


def _w_spec(shape):
  return pl.BlockSpec(shape, lambda i: (0,) * len(shape))


def _front1(x, wp, bp, wq, bq, wk, bk, wv, bv):
  n, in_dim = x.shape
  r = _TC_R
  grid = (n // r,)
  f32 = jnp.float32
  return pl.pallas_call(
      _front1_body,
      grid=grid,
      in_specs=[
          pl.BlockSpec((r, in_dim), lambda i: (i, 0)),
          _w_spec((in_dim, HID)), _w_spec((1, HID)),
          _w_spec((HID, HID)), _w_spec((1, HID)),
          _w_spec((HID, HID)), _w_spec((1, HID)),
          _w_spec((HID, HID)), _w_spec((1, HID)),
      ],
      out_specs=[
          pl.BlockSpec((r, HID), lambda i: (i, 0)),
          pl.BlockSpec((HEADS, r, D), lambda i: (0, i, 0)),
          pl.BlockSpec((HEADS, r, D), lambda i: (0, i, 0)),
          pl.BlockSpec((HEADS, r, D), lambda i: (0, i, 0)),
      ],
      out_shape=[
          jax.ShapeDtypeStruct((n, HID), f32),
          jax.ShapeDtypeStruct((HEADS, n, D), f32),
          jax.ShapeDtypeStruct((HEADS, n, D), f32),
          jax.ShapeDtypeStruct((HEADS, n, D), f32),
      ],
  )(x, wp, bp, wq, bq, wk, bk, wv, bv)


def _front2(h, wq, bq, wk, bk, wv, bv):
  n = h.shape[0]
  r = _TC_R
  f32 = jnp.float32
  return pl.pallas_call(
      _front2_body,
      grid=(n // r,),
      in_specs=[
          pl.BlockSpec((r, HID), lambda i: (i, 0)),
          _w_spec((HID, HID)), _w_spec((1, HID)),
          _w_spec((HID, HID)), _w_spec((1, HID)),
          _w_spec((HID, HID)), _w_spec((1, HID)),
      ],
      out_specs=[
          pl.BlockSpec((HEADS, r, D), lambda i: (0, i, 0)),
          pl.BlockSpec((HEADS, r, D), lambda i: (0, i, 0)),
          pl.BlockSpec((HEADS, r, D), lambda i: (0, i, 0)),
      ],
      out_shape=[
          jax.ShapeDtypeStruct((HEADS, n, D), f32),
          jax.ShapeDtypeStruct((HEADS, n, D), f32),
          jax.ShapeDtypeStruct((HEADS, n, D), f32),
      ],
  )(h, wq, bq, wk, bk, wv, bv)


def _back(nd, h, wo, bo, skip, relu):
  n = h.shape[0]
  r = _TC_R
  return pl.pallas_call(
      functools.partial(_back_body, relu=relu),
      grid=(n // r,),
      in_specs=[
          pl.BlockSpec((HEADS, r, 24), lambda i: (0, i, 0)),
          pl.BlockSpec((r, HID), lambda i: (i, 0)),
          _w_spec((HID, HID)), _w_spec((1, HID)),
          _w_spec((1, 1)),
      ],
      out_specs=pl.BlockSpec((r, HID), lambda i: (i, 0)),
      out_shape=jax.ShapeDtypeStruct((n, HID), jnp.float32),
  )(nd, h, wo, bo, skip)


_C = 400
_ZR = 136


@functools.cache
def _make_sc_et(n, e):
  assert e % (_C * 16) == 0
  m = e // (_C * 16)
  assert m % 2 == 1 and m >= 3
  mp = (m - 1) // 2
  groups = _C // 16
  rpt = ((n // 16 + 7) // 8) * 8
  n_pad = 16 * rpt
  zr = _ZR if rpt % _ZR == 0 else 8
  f32 = jnp.float32
  i32 = jnp.int32
  mesh = plsc.VectorSubcoreMesh(core_axis_name="c", subcore_axis_name="s")

  @functools.partial(
      pl.kernel,
      mesh=mesh,
      out_type=(jax.ShapeDtypeStruct((HEADS, n_pad, 24), f32),
                jax.ShapeDtypeStruct((HEADS, e), f32)),
      compiler_params=pltpu.CompilerParams(
          use_tc_tiling_on_sc=False, needs_layout_passes=False),
      scratch_types=[
          pltpu.VMEM((2, _C), i32),
          pltpu.VMEM((2, _C), i32),
          pltpu.VMEM((2, _C), i32),
          pltpu.VMEM((2, _C, 16), f32),
          pltpu.VMEM((2, _C, 16), f32),
          pltpu.VMEM((2, _C), f32),
          pltpu.VMEM((272,), f32),
          pltpu.VMEM((16,), f32),
          pltpu.VMEM((2, _C, 24), f32),
          pltpu.VMEM((zr, 24), f32),
          pltpu.VMEM((16, 16), f32),
          pltpu.VMEM_SHARED((n_pad, 24), f32),
          pltpu.VMEM_SHARED((16, 16), f32),
          pltpu.SemaphoreType.DMA,
          pltpu.SemaphoreType.DMA,
          pltpu.SemaphoreType.DMA,
          pltpu.SemaphoreType.DMA,
          pltpu.SemaphoreType.DMA,
          pltpu.SemaphoreType.DMA,
      ],
  )
  def sc_et(src_hbm, dst_hbm, ktab, qtab, vtab, nd_out, alpha_out,
            sidx2, didx2, dscat2, gbuf2, qbuf2, abuf2, scr, mbuf, msgbuf2,
            zbuf, tred, acc, tmax, semk0, semk1, semq0, semq1, semw0, semw1):
    c = lax.axis_index("c")
    s = lax.axis_index("s")
    iota = lax.iota(i32, 16)
    idx_pitch0 = iota * 17
    idx_15 = (iota * 0) + 15
    zero16 = jnp.zeros((16,), f32)
    sidx = (sidx2.at[0], sidx2.at[1])
    didx = (didx2.at[0], didx2.at[1])
    dscat = (dscat2.at[0], dscat2.at[1])
    gbuf = (gbuf2.at[0], gbuf2.at[1])
    qbuf = (qbuf2.at[0], qbuf2.at[1])
    abuf = (abuf2.at[0], abuf2.at[1])
    msgbuf = (msgbuf2.at[0], msgbuf2.at[1])
    semk = (semk0, semk1)
    semq = (semq0, semq1)
    semw = (semw0, semw1)

    def chunk_base(j):
      return (s * m + j) * _C

    def zrow(r, _):
      zbuf[r, pl.ds(8, 16)] = zero16
      zbuf[r, pl.ds(0, 16)] = zero16
      return _
    lax.fori_loop(0, zr, zrow, None)

    def zcopy(z, _):
      pltpu.sync_copy(zbuf, acc.at[pl.ds(s * rpt + z * zr, zr), :])
      return _
    lax.fori_loop(0, rpt // zr, zcopy, None)

    for p in range(2):
      def zmrow(r, _, p=p):
        msgbuf[p][r, pl.ds(8, 16)] = zero16
        return _
      lax.fori_loop(0, _C, zmrow, None)
    plsc.subcore_barrier()

    mbuf[...] = jnp.full((16,), -1e30, f32)

    def fire_a(j, p):
      base = chunk_base(j)
      pltpu.sync_copy(src_hbm.at[pl.ds(base, _C)], sidx[p])
      pltpu.sync_copy(dst_hbm.at[pl.ds(base, _C)], didx[p])
      pltpu.async_copy(ktab.at[c].at[sidx[p]], gbuf[p], semk[p])
      pltpu.async_copy(qtab.at[c].at[didx[p]], qbuf[p], semq[p])

    def comp_a(j, p, wait_w):
      base = chunk_base(j)
      pltpu.make_async_copy(ktab.at[c].at[sidx[p]], gbuf[p], semk[p]).wait()
      pltpu.make_async_copy(qtab.at[c].at[didx[p]], qbuf[p], semq[p]).wait()
      def _wait_w():
        pltpu.make_async_copy(
            abuf[p], alpha_out.at[c, pl.ds(base, _C)], semw[p]).wait()
      if wait_w is True:
        _wait_w()
      elif wait_w is not None:
        pl.when(wait_w)(_wait_w)

      def group_a(g, _):
        for ee in range(16):
          i = g * 16 + ee
          scr[pl.ds(17 * ee, 16)] = gbuf[p][i] * qbuf[p][i]
        avec = jnp.zeros((16,), f32)
        for d in range(16):
          avec = avec + plsc.load_gather(scr, [idx_pitch0 + d])
        abuf[p][pl.ds(g * 16, 16)] = avec
        mbuf[...] = jnp.maximum(mbuf[...], avec)
        return _
      lax.fori_loop(0, groups, group_a, None)
      pltpu.async_copy(abuf[p], alpha_out.at[c, pl.ds(base, _C)], semw[p])

    fire_a(0, 0)
    fire_a(1, 1)

    def body_a(j2, _):
      j0 = 2 * j2
      comp_a(j0, 0, j2 > 0)
      fire_a(j0 + 2, 0)
      comp_a(j0 + 1, 1, j2 > 0)
      @pl.when(j2 < mp - 1)
      def _():
        fire_a(j0 + 3, 1)
      return _
    lax.fori_loop(0, mp, body_a, None)
    comp_a(m - 1, 0, True)
    pltpu.make_async_copy(
        abuf[0], alpha_out.at[c, pl.ds(chunk_base(m - 1), _C)], semw[0]).wait()
    pltpu.make_async_copy(
        abuf[1], alpha_out.at[c, pl.ds(chunk_base(m - 2), _C)], semw[1]).wait()

    pltpu.sync_copy(mbuf, tmax.at[s])
    plsc.subcore_barrier()
    pltpu.sync_copy(tmax, tred)
    mx = jnp.full((16,), -1e30, f32)
    for r in range(16):
      mx = jnp.maximum(mx, tred[r])
    scr[pl.ds(0, 16)] = plsc.cummax(mx)
    svec = plsc.load_gather(scr, [idx_15])

    def fire_b(j, p):
      base = chunk_base(j)
      pltpu.sync_copy(src_hbm.at[pl.ds(base, _C)], sidx[p])
      pltpu.sync_copy(dst_hbm.at[pl.ds(base, _C)], didx[p])
      pltpu.async_copy(vtab.at[c].at[sidx[p]], gbuf[p], semk[p])
      pltpu.async_copy(alpha_out.at[c, pl.ds(base, _C)], abuf[p], semq[p])

    def comp_b(j, p, wait_w):
      base = chunk_base(j)
      pltpu.make_async_copy(vtab.at[c].at[sidx[p]], gbuf[p], semk[p]).wait()
      pltpu.make_async_copy(
          alpha_out.at[c, pl.ds(base, _C)], abuf[p], semq[p]).wait()
      def _wait_w():
        pltpu.make_async_copy(msgbuf[p], acc.at[dscat[p]], semw[p]).wait()
      if wait_w is True:
        _wait_w()
      elif wait_w is not None:
        pl.when(wait_w)(_wait_w)

      def dcopy(i, _):
        dscat[p][pl.ds(i * 16, 16)] = didx[p][pl.ds(i * 16, 16)]
        return _
      lax.fori_loop(0, _C // 16, dcopy, None)

      def group_b(g, _):
        evec = iota + g * 16
        wvec = jnp.exp(abuf[p][pl.ds(g * 16, 16)] - svec)
        scr[pl.ds(0, 16)] = wvec
        plsc.store_scatter(msgbuf[p], [evec, (iota * 0) + 16], wvec)
        for ee in range(16):
          i = g * 16 + ee
          sp = plsc.load_gather(scr, [(iota * 0) + ee])
          msgbuf[p][i, pl.ds(0, 16)] = sp * gbuf[p][i]
        return _
      lax.fori_loop(0, groups, group_b, None)
      pltpu.async_copy(msgbuf[p], acc.at[dscat[p]], semw[p], add=True)

    fire_b(0, 0)
    fire_b(1, 1)

    def body_b(j2, _):
      j0 = 2 * j2
      comp_b(j0, 0, j2 > 0)
      fire_b(j0 + 2, 0)
      comp_b(j0 + 1, 1, j2 > 0)
      @pl.when(j2 < mp - 1)
      def _():
        fire_b(j0 + 3, 1)
      return _
    lax.fori_loop(0, mp, body_b, None)
    comp_b(m - 1, 0, True)
    pltpu.make_async_copy(msgbuf[0], acc.at[dscat[0]], semw[0]).wait()
    pltpu.make_async_copy(msgbuf[1], acc.at[dscat[1]], semw[1]).wait()

    plsc.subcore_barrier()
    pltpu.sync_copy(acc.at[pl.ds(s * rpt, rpt), :],
                    nd_out.at[c, pl.ds(s * rpt, rpt), :])

  return sc_et


def _eff_weights(cp, nt, rel):
  scale = 1.0 / math.sqrt(D)
  a = cp["a_" + rel]
  m = cp["m_" + rel]
  p = cp["p_" + rel]
  za = jnp.zeros((HID, HID), jnp.float32)
  za = za.at[:D, :D].set(a[0] * (p[0] * scale))
  za = za.at[D:, D:].set(a[1] * (p[1] * scale))
  zm = jnp.zeros((HID, HID), jnp.float32)
  zm = zm.at[:D, :D].set(m[0])
  zm = zm.at[D:, D:].set(m[1])
  wk = cp["Wk_" + nt] @ za
  bk = (cp["bk_" + nt] @ za).reshape(1, HID)
  wv = cp["Wv_" + nt] @ zm
  bv = (cp["bv_" + nt] @ zm).reshape(1, HID)
  wq = cp["Wq_" + nt]
  bq = cp["bq_" + nt].reshape(1, HID)
  return wq, bq, wk, bk, wv, bv


def kernel(x_user, x_item, edge_index_user_item, edge_index_item_user, params):
  n = x_user.shape[0]
  e = edge_index_user_item.shape[1]
  sc_et = _make_sc_et(n, e)

  src_ui = edge_index_user_item[0]
  dst_ui = edge_index_user_item[1]
  src_iu = edge_index_item_user[0]
  dst_iu = edge_index_item_user[1]

  c1, c2 = params["c1"], params["c2"]
  eff1_u = _eff_weights(c1, "user", "to")
  eff1_i = _eff_weights(c1, "item", "rev")
  eff2_u = _eff_weights(c2, "user", "to")
  eff2_i = _eff_weights(c2, "item", "rev")

  h_u, qt_u, kt_u, vt_u = _front1(
      x_user, params["Wp_user"], params["bp_user"].reshape(1, HID), *eff1_u)
  h_i, qt_i, kt_i, vt_i = _front1(
      x_item, params["Wp_item"], params["bp_item"].reshape(1, HID), *eff1_i)

  nd_to, _ = sc_et(src_ui, dst_ui, kt_u, qt_i, vt_u)
  nd_rev, _ = sc_et(src_iu, dst_iu, kt_i, qt_u, vt_i)

  h2_u = _back(nd_rev, h_u, c1["Wo_user"], c1["bo_user"].reshape(1, HID),
               c1["skip_user"].reshape(1, 1), relu=True)
  h2_i = _back(nd_to, h_i, c1["Wo_item"], c1["bo_item"].reshape(1, HID),
               c1["skip_item"].reshape(1, 1), relu=True)

  qt2_u, kt2_u, vt2_u = _front2(h2_u, *eff2_u)
  qt2_i, kt2_i, vt2_i = _front2(h2_i, *eff2_i)

  nd_to2, _ = sc_et(src_ui, dst_ui, kt2_u, qt2_i, vt2_u)
  nd_rev2, _ = sc_et(src_iu, dst_iu, kt2_i, qt2_u, vt2_i)

  out_u = _back(nd_rev2, h2_u, c2["Wo_user"], c2["bo_user"].reshape(1, HID),
                c2["skip_user"].reshape(1, 1), relu=False)
  out_i = _back(nd_to2, h2_i, c2["Wo_item"], c2["bo_item"].reshape(1, HID),
                c2["skip_item"].reshape(1, 1), relu=False)
  return out_u, out_i


# speedup vs baseline: 112.4094x; 1.2899x over previous
import functools
import math

import jax
import jax.numpy as jnp
from jax import lax
from jax.experimental import pallas as pl
from jax.experimental.pallas import tpu as pltpu
from jax.experimental.pallas import tpu_sc as plsc

HEADS = 2
D = 16
HID = HEADS * D
EPS = 1e-16


_TC_R = 1000


def _front1_body(x_ref, wp_ref, bp_ref, wq_ref, bq_ref, wk_ref, bk_ref,
                 wv_ref, bv_ref, h_ref, qt_ref, kt_ref, vt_ref):
  x = x_ref[...]
  h = jnp.maximum(
      jnp.dot(x, wp_ref[...], preferred_element_type=jnp.float32)
      + bp_ref[...], 0.0)
  q = jnp.dot(h, wq_ref[...], preferred_element_type=jnp.float32) + bq_ref[...]
  k = jnp.dot(h, wk_ref[...], preferred_element_type=jnp.float32) + bk_ref[...]
  v = jnp.dot(h, wv_ref[...], preferred_element_type=jnp.float32) + bv_ref[...]
  h_ref[...] = h
  qt_ref[0] = q[:, :D]
  qt_ref[1] = q[:, D:]
  kt_ref[0] = k[:, :D]
  kt_ref[1] = k[:, D:]
  vt_ref[0] = v[:, :D]
  vt_ref[1] = v[:, D:]


def _front2_body(h_ref, wq_ref, bq_ref, wk_ref, bk_ref, wv_ref, bv_ref,
                 qt_ref, kt_ref, vt_ref):
  h = h_ref[...]
  q = jnp.dot(h, wq_ref[...], preferred_element_type=jnp.float32) + bq_ref[...]
  k = jnp.dot(h, wk_ref[...], preferred_element_type=jnp.float32) + bk_ref[...]
  v = jnp.dot(h, wv_ref[...], preferred_element_type=jnp.float32) + bv_ref[...]
  qt_ref[0] = q[:, :D]
  qt_ref[1] = q[:, D:]
  kt_ref[0] = k[:, :D]
  kt_ref[1] = k[:, D:]
  vt_ref[0] = v[:, :D]
  vt_ref[1] = v[:, D:]


def _back_body(nd_ref, h_ref, wo_ref, bo_ref, skip_ref, out_ref, *, relu):
  num0 = nd_ref[0, :, :D]
  den0 = nd_ref[0, :, D:D + 1]
  num1 = nd_ref[1, :, :D]
  den1 = nd_ref[1, :, D:D + 1]
  agg = jnp.concatenate([num0 / (den0 + EPS), num1 / (den1 + EPS)], axis=-1)
  o = 0.5 * agg * (1.0 + lax.erf(agg * (1.0 / math.sqrt(2.0))))
  o = jnp.dot(o, wo_ref[...], preferred_element_type=jnp.float32) + bo_ref[...]
  beta = jax.nn.sigmoid(skip_ref[0, 0])
  out = beta * o + (1.0 - beta) * h_ref[...]
  if relu:
    out = jnp.maximum(out, 0.0)
  out_ref[...] = out


def _w_spec(shape):
  return pl.BlockSpec(shape, lambda i: (0,) * len(shape))


def _front1(x, wp, bp, wq, bq, wk, bk, wv, bv):
  n, in_dim = x.shape
  r = _TC_R
  grid = (n // r,)
  f32 = jnp.float32
  return pl.pallas_call(
      _front1_body,
      grid=grid,
      in_specs=[
          pl.BlockSpec((r, in_dim), lambda i: (i, 0)),
          _w_spec((in_dim, HID)), _w_spec((1, HID)),
          _w_spec((HID, HID)), _w_spec((1, HID)),
          _w_spec((HID, HID)), _w_spec((1, HID)),
          _w_spec((HID, HID)), _w_spec((1, HID)),
      ],
      out_specs=[
          pl.BlockSpec((r, HID), lambda i: (i, 0)),
          pl.BlockSpec((HEADS, r, D), lambda i: (0, i, 0)),
          pl.BlockSpec((HEADS, r, D), lambda i: (0, i, 0)),
          pl.BlockSpec((HEADS, r, D), lambda i: (0, i, 0)),
      ],
      out_shape=[
          jax.ShapeDtypeStruct((n, HID), f32),
          jax.ShapeDtypeStruct((HEADS, n, D), f32),
          jax.ShapeDtypeStruct((HEADS, n, D), f32),
          jax.ShapeDtypeStruct((HEADS, n, D), f32),
      ],
  )(x, wp, bp, wq, bq, wk, bk, wv, bv)


def _front2(h, wq, bq, wk, bk, wv, bv):
  n = h.shape[0]
  r = _TC_R
  f32 = jnp.float32
  return pl.pallas_call(
      _front2_body,
      grid=(n // r,),
      in_specs=[
          pl.BlockSpec((r, HID), lambda i: (i, 0)),
          _w_spec((HID, HID)), _w_spec((1, HID)),
          _w_spec((HID, HID)), _w_spec((1, HID)),
          _w_spec((HID, HID)), _w_spec((1, HID)),
      ],
      out_specs=[
          pl.BlockSpec((HEADS, r, D), lambda i: (0, i, 0)),
          pl.BlockSpec((HEADS, r, D), lambda i: (0, i, 0)),
          pl.BlockSpec((HEADS, r, D), lambda i: (0, i, 0)),
      ],
      out_shape=[
          jax.ShapeDtypeStruct((HEADS, n, D), f32),
          jax.ShapeDtypeStruct((HEADS, n, D), f32),
          jax.ShapeDtypeStruct((HEADS, n, D), f32),
      ],
  )(h, wq, bq, wk, bk, wv, bv)


def _back(nd, h, wo, bo, skip, relu):
  n = h.shape[0]
  r = _TC_R
  return pl.pallas_call(
      functools.partial(_back_body, relu=relu),
      grid=(n // r,),
      in_specs=[
          pl.BlockSpec((HEADS, r, 24), lambda i: (0, i, 0)),
          pl.BlockSpec((r, HID), lambda i: (i, 0)),
          _w_spec((HID, HID)), _w_spec((1, HID)),
          _w_spec((1, 1)),
      ],
      out_specs=pl.BlockSpec((r, HID), lambda i: (i, 0)),
      out_shape=jax.ShapeDtypeStruct((n, HID), jnp.float32),
  )(nd, h, wo, bo, skip)


_C = 400
_ZR = 136


@functools.cache
def _make_sc_et(n, e):
  assert e % (_C * 16) == 0
  m = e // (_C * 16)
  assert m % 2 == 1 and m >= 3
  mp = (m - 1) // 2
  groups = _C // 16
  rpt = ((n // 16 + 7) // 8) * 8
  n_pad = 16 * rpt
  zr = _ZR if rpt % _ZR == 0 else 8
  f32 = jnp.float32
  i32 = jnp.int32
  mesh = plsc.VectorSubcoreMesh(core_axis_name="c", subcore_axis_name="s")

  @functools.partial(
      pl.kernel,
      mesh=mesh,
      out_type=(jax.ShapeDtypeStruct((HEADS, n_pad, 24), f32),
                jax.ShapeDtypeStruct((HEADS, e), f32)),
      compiler_params=pltpu.CompilerParams(
          use_tc_tiling_on_sc=False, needs_layout_passes=False),
      scratch_types=[
          pltpu.VMEM((2, _C), i32),
          pltpu.VMEM((2, _C), i32),
          pltpu.VMEM((2, _C), i32),
          pltpu.VMEM((2, _C, 16), f32),
          pltpu.VMEM((2, _C, 16), f32),
          pltpu.VMEM((2, _C), f32),
          pltpu.VMEM((272,), f32),
          pltpu.VMEM((16,), f32),
          pltpu.VMEM((2, _C, 24), f32),
          pltpu.VMEM((zr, 24), f32),
          pltpu.VMEM((16, 16), f32),
          pltpu.VMEM_SHARED((n_pad, 24), f32),
          pltpu.VMEM_SHARED((16, 16), f32),
          pltpu.SemaphoreType.DMA,
          pltpu.SemaphoreType.DMA,
          pltpu.SemaphoreType.DMA,
          pltpu.SemaphoreType.DMA,
          pltpu.SemaphoreType.DMA,
          pltpu.SemaphoreType.DMA,
          pltpu.SemaphoreType.DMA,
          pltpu.SemaphoreType.DMA,
      ],
  )
  def sc_et(src_hbm, dst_hbm, ktab, qtab, vtab, nd_out, alpha_out,
            sidx2, didx2, dscat2, gbuf2, qbuf2, abuf2, scr, mbuf, msgbuf2,
            zbuf, tred, acc, tmax, semk0, semk1, semq0, semq1, semw0, semw1,
            semi0, semi1):
    c = lax.axis_index("c")
    s = lax.axis_index("s")
    iota = lax.iota(i32, 16)
    idx_pitch0 = iota * 17
    idx_15 = (iota * 0) + 15
    zero16 = jnp.zeros((16,), f32)
    sidx = (sidx2.at[0], sidx2.at[1])
    didx = (didx2.at[0], didx2.at[1])
    dscat = (dscat2.at[0], dscat2.at[1])
    gbuf = (gbuf2.at[0], gbuf2.at[1])
    qbuf = (qbuf2.at[0], qbuf2.at[1])
    abuf = (abuf2.at[0], abuf2.at[1])
    msgbuf = (msgbuf2.at[0], msgbuf2.at[1])
    semk = (semk0, semk1)
    semq = (semq0, semq1)
    semw = (semw0, semw1)
    semi = (semi0, semi1)

    def chunk_base(j):
      return (s * m + j) * _C

    def fire_idx(j, p):
      base = chunk_base(j)
      pltpu.async_copy(src_hbm.at[pl.ds(base, _C)], sidx[p], semi[p])
      pltpu.async_copy(dst_hbm.at[pl.ds(base, _C)], didx[p], semi[p])

    def wait_idx(j, p):
      base = chunk_base(j)
      pltpu.make_async_copy(src_hbm.at[pl.ds(base, _C)], sidx[p],
                            semi[p]).wait()
      pltpu.make_async_copy(dst_hbm.at[pl.ds(base, _C)], didx[p],
                            semi[p]).wait()

    def zrow(r, _):
      zbuf[r, pl.ds(8, 16)] = zero16
      zbuf[r, pl.ds(0, 16)] = zero16
      return _
    lax.fori_loop(0, zr, zrow, None)

    def zcopy(z, _):
      pltpu.sync_copy(zbuf, acc.at[pl.ds(s * rpt + z * zr, zr), :])
      return _
    lax.fori_loop(0, rpt // zr, zcopy, None)

    for p in range(2):
      def zmrow(r, _, p=p):
        msgbuf[p][r, pl.ds(8, 16)] = zero16
        return _
      lax.fori_loop(0, _C, zmrow, None)
    plsc.subcore_barrier()

    mbuf[...] = jnp.full((16,), -1e30, f32)

    def fire_a(j, p):
      wait_idx(j, p)
      pltpu.async_copy(ktab.at[c].at[sidx[p]], gbuf[p], semk[p])
      pltpu.async_copy(qtab.at[c].at[didx[p]], qbuf[p], semq[p])

    def comp_a(j, p, wait_w, next_j=None, next_guard=None):
      base = chunk_base(j)
      pltpu.make_async_copy(ktab.at[c].at[sidx[p]], gbuf[p], semk[p]).wait()
      pltpu.make_async_copy(qtab.at[c].at[didx[p]], qbuf[p], semq[p]).wait()
      def _wait_w():
        pltpu.make_async_copy(
            abuf[p], alpha_out.at[c, pl.ds(base, _C)], semw[p]).wait()
      if wait_w is True:
        _wait_w()
      elif wait_w is not None:
        pl.when(wait_w)(_wait_w)
      if next_j is not None:
        if next_guard is None:
          fire_idx(next_j, p)
        else:
          pl.when(next_guard)(lambda: fire_idx(next_j, p))

      def group_a(g, _):
        for ee in range(16):
          i = g * 16 + ee
          scr[pl.ds(17 * ee, 16)] = gbuf[p][i] * qbuf[p][i]
        avec = jnp.zeros((16,), f32)
        for d in range(16):
          avec = avec + plsc.load_gather(scr, [idx_pitch0 + d])
        abuf[p][pl.ds(g * 16, 16)] = avec
        mbuf[...] = jnp.maximum(mbuf[...], avec)
        return _
      lax.fori_loop(0, groups, group_a, None)
      pltpu.async_copy(abuf[p], alpha_out.at[c, pl.ds(base, _C)], semw[p])

    fire_idx(0, 0)
    fire_idx(1, 1)
    fire_a(0, 0)
    fire_a(1, 1)

    def body_a(j2, _):
      j0 = 2 * j2
      comp_a(j0, 0, j2 > 0, next_j=j0 + 2)
      fire_a(j0 + 2, 0)
      comp_a(j0 + 1, 1, j2 > 0, next_j=j0 + 3, next_guard=j2 < mp - 1)
      @pl.when(j2 < mp - 1)
      def _():
        fire_a(j0 + 3, 1)
      return _
    lax.fori_loop(0, mp, body_a, None)
    comp_a(m - 1, 0, True)
    pltpu.make_async_copy(
        abuf[0], alpha_out.at[c, pl.ds(chunk_base(m - 1), _C)], semw[0]).wait()
    pltpu.make_async_copy(
        abuf[1], alpha_out.at[c, pl.ds(chunk_base(m - 2), _C)], semw[1]).wait()

    pltpu.sync_copy(mbuf, tmax.at[s])
    plsc.subcore_barrier()
    pltpu.sync_copy(tmax, tred)
    mx = jnp.full((16,), -1e30, f32)
    for r in range(16):
      mx = jnp.maximum(mx, tred[r])
    scr[pl.ds(0, 16)] = plsc.cummax(mx)
    svec = plsc.load_gather(scr, [idx_15])

    def fire_b(j, p):
      base = chunk_base(j)
      wait_idx(j, p)
      pltpu.async_copy(vtab.at[c].at[sidx[p]], gbuf[p], semk[p])
      pltpu.async_copy(alpha_out.at[c, pl.ds(base, _C)], abuf[p], semq[p])

    def comp_b(j, p, wait_w, next_j=None, next_guard=None):
      base = chunk_base(j)
      pltpu.make_async_copy(vtab.at[c].at[sidx[p]], gbuf[p], semk[p]).wait()
      pltpu.make_async_copy(
          alpha_out.at[c, pl.ds(base, _C)], abuf[p], semq[p]).wait()
      def _wait_w():
        pltpu.make_async_copy(msgbuf[p], acc.at[dscat[p]], semw[p]).wait()
      if wait_w is True:
        _wait_w()
      elif wait_w is not None:
        pl.when(wait_w)(_wait_w)

      def dcopy(i, _):
        dscat[p][pl.ds(i * 16, 16)] = didx[p][pl.ds(i * 16, 16)]
        return _
      lax.fori_loop(0, _C // 16, dcopy, None)
      if next_j is not None:
        if next_guard is None:
          fire_idx(next_j, p)
        else:
          pl.when(next_guard)(lambda: fire_idx(next_j, p))

      def group_b(g, _):
        evec = iota + g * 16
        wvec = jnp.exp(abuf[p][pl.ds(g * 16, 16)] - svec)
        scr[pl.ds(0, 16)] = wvec
        plsc.store_scatter(msgbuf[p], [evec, (iota * 0) + 16], wvec)
        for ee in range(16):
          i = g * 16 + ee
          sp = plsc.load_gather(scr, [(iota * 0) + ee])
          msgbuf[p][i, pl.ds(0, 16)] = sp * gbuf[p][i]
        return _
      lax.fori_loop(0, groups, group_b, None)
      pltpu.async_copy(msgbuf[p], acc.at[dscat[p]], semw[p], add=True)

    fire_idx(0, 0)
    fire_idx(1, 1)
    fire_b(0, 0)
    fire_b(1, 1)

    def body_b(j2, _):
      j0 = 2 * j2
      comp_b(j0, 0, j2 > 0, next_j=j0 + 2)
      fire_b(j0 + 2, 0)
      comp_b(j0 + 1, 1, j2 > 0, next_j=j0 + 3, next_guard=j2 < mp - 1)
      @pl.when(j2 < mp - 1)
      def _():
        fire_b(j0 + 3, 1)
      return _
    lax.fori_loop(0, mp, body_b, None)
    comp_b(m - 1, 0, True)
    pltpu.make_async_copy(msgbuf[0], acc.at[dscat[0]], semw[0]).wait()
    pltpu.make_async_copy(msgbuf[1], acc.at[dscat[1]], semw[1]).wait()

    plsc.subcore_barrier()
    pltpu.sync_copy(acc.at[pl.ds(s * rpt, rpt), :],
                    nd_out.at[c, pl.ds(s * rpt, rpt), :])

  return sc_et


def _eff_weights(cp, nt, rel):
  scale = 1.0 / math.sqrt(D)
  a = cp["a_" + rel]
  m = cp["m_" + rel]
  p = cp["p_" + rel]
  za = jnp.zeros((HID, HID), jnp.float32)
  za = za.at[:D, :D].set(a[0] * (p[0] * scale))
  za = za.at[D:, D:].set(a[1] * (p[1] * scale))
  zm = jnp.zeros((HID, HID), jnp.float32)
  zm = zm.at[:D, :D].set(m[0])
  zm = zm.at[D:, D:].set(m[1])
  wk = cp["Wk_" + nt] @ za
  bk = (cp["bk_" + nt] @ za).reshape(1, HID)
  wv = cp["Wv_" + nt] @ zm
  bv = (cp["bv_" + nt] @ zm).reshape(1, HID)
  wq = cp["Wq_" + nt]
  bq = cp["bq_" + nt].reshape(1, HID)
  return wq, bq, wk, bk, wv, bv


def kernel(x_user, x_item, edge_index_user_item, edge_index_item_user, params):
  n = x_user.shape[0]
  e = edge_index_user_item.shape[1]
  sc_et = _make_sc_et(n, e)

  src_ui = edge_index_user_item[0]
  dst_ui = edge_index_user_item[1]
  src_iu = edge_index_item_user[0]
  dst_iu = edge_index_item_user[1]

  c1, c2 = params["c1"], params["c2"]
  eff1_u = _eff_weights(c1, "user", "to")
  eff1_i = _eff_weights(c1, "item", "rev")
  eff2_u = _eff_weights(c2, "user", "to")
  eff2_i = _eff_weights(c2, "item", "rev")

  h_u, qt_u, kt_u, vt_u = _front1(
      x_user, params["Wp_user"], params["bp_user"].reshape(1, HID), *eff1_u)
  h_i, qt_i, kt_i, vt_i = _front1(
      x_item, params["Wp_item"], params["bp_item"].reshape(1, HID), *eff1_i)

  nd_to, _ = sc_et(src_ui, dst_ui, kt_u, qt_i, vt_u)
  nd_rev, _ = sc_et(src_iu, dst_iu, kt_i, qt_u, vt_i)

  h2_u = _back(nd_rev, h_u, c1["Wo_user"], c1["bo_user"].reshape(1, HID),
               c1["skip_user"].reshape(1, 1), relu=True)
  h2_i = _back(nd_to, h_i, c1["Wo_item"], c1["bo_item"].reshape(1, HID),
               c1["skip_item"].reshape(1, 1), relu=True)

  qt2_u, kt2_u, vt2_u = _front2(h2_u, *eff2_u)
  qt2_i, kt2_i, vt2_i = _front2(h2_i, *eff2_i)

  nd_to2, _ = sc_et(src_ui, dst_ui, kt2_u, qt2_i, vt2_u)
  nd_rev2, _ = sc_et(src_iu, dst_iu, kt2_i, qt2_u, vt2_i)

  out_u = _back(nd_rev2, h2_u, c2["Wo_user"], c2["bo_user"].reshape(1, HID),
                c2["skip_user"].reshape(1, 1), relu=False)
  out_i = _back(nd_to2, h2_i, c2["Wo_item"], c2["bo_item"].reshape(1, HID),
                c2["skip_item"].reshape(1, 1), relu=False)
  return out_u, out_i


# speedup vs baseline: 112.9958x; 1.0052x over previous
import functools
import math

import jax
import jax.numpy as jnp
from jax import lax
from jax.experimental import pallas as pl
from jax.experimental.pallas import tpu as pltpu
from jax.experimental.pallas import tpu_sc as plsc

HEADS = 2
D = 16
HID = HEADS * D
EPS = 1e-16


_TC_R = 1000


def _front1_body(x_ref, wp_ref, bp_ref, wq_ref, bq_ref, wk_ref, bk_ref,
                 wv_ref, bv_ref, h_ref, qt_ref, kt_ref, vt_ref):
  x = x_ref[...]
  h = jnp.maximum(
      jnp.dot(x, wp_ref[...], preferred_element_type=jnp.float32)
      + bp_ref[...], 0.0)
  q = jnp.dot(h, wq_ref[...], preferred_element_type=jnp.float32) + bq_ref[...]
  k = jnp.dot(h, wk_ref[...], preferred_element_type=jnp.float32) + bk_ref[...]
  v = jnp.dot(h, wv_ref[...], preferred_element_type=jnp.float32) + bv_ref[...]
  h_ref[...] = h
  qt_ref[0] = q[:, :D]
  qt_ref[1] = q[:, D:]
  kt_ref[0] = k[:, :D]
  kt_ref[1] = k[:, D:]
  vt_ref[0] = v[:, :D]
  vt_ref[1] = v[:, D:]


def _front2_body(h_ref, wq_ref, bq_ref, wk_ref, bk_ref, wv_ref, bv_ref,
                 qt_ref, kt_ref, vt_ref):
  h = h_ref[...]
  q = jnp.dot(h, wq_ref[...], preferred_element_type=jnp.float32) + bq_ref[...]
  k = jnp.dot(h, wk_ref[...], preferred_element_type=jnp.float32) + bk_ref[...]
  v = jnp.dot(h, wv_ref[...], preferred_element_type=jnp.float32) + bv_ref[...]
  qt_ref[0] = q[:, :D]
  qt_ref[1] = q[:, D:]
  kt_ref[0] = k[:, :D]
  kt_ref[1] = k[:, D:]
  vt_ref[0] = v[:, :D]
  vt_ref[1] = v[:, D:]


def _back_body(nd_ref, h_ref, wo_ref, bo_ref, skip_ref, out_ref, *, relu):
  num0 = nd_ref[0, :, :D]
  den0 = nd_ref[0, :, D:D + 1]
  num1 = nd_ref[1, :, :D]
  den1 = nd_ref[1, :, D:D + 1]
  agg = jnp.concatenate([num0 / (den0 + EPS), num1 / (den1 + EPS)], axis=-1)
  o = 0.5 * agg * (1.0 + lax.erf(agg * (1.0 / math.sqrt(2.0))))
  o = jnp.dot(o, wo_ref[...], preferred_element_type=jnp.float32) + bo_ref[...]
  beta = jax.nn.sigmoid(skip_ref[0, 0])
  out = beta * o + (1.0 - beta) * h_ref[...]
  if relu:
    out = jnp.maximum(out, 0.0)
  out_ref[...] = out


def _w_spec(shape):
  return pl.BlockSpec(shape, lambda i: (0,) * len(shape))


def _front1(x, wp, bp, wq, bq, wk, bk, wv, bv):
  n, in_dim = x.shape
  r = _TC_R
  grid = (n // r,)
  f32 = jnp.float32
  return pl.pallas_call(
      _front1_body,
      grid=grid,
      in_specs=[
          pl.BlockSpec((r, in_dim), lambda i: (i, 0)),
          _w_spec((in_dim, HID)), _w_spec((1, HID)),
          _w_spec((HID, HID)), _w_spec((1, HID)),
          _w_spec((HID, HID)), _w_spec((1, HID)),
          _w_spec((HID, HID)), _w_spec((1, HID)),
      ],
      out_specs=[
          pl.BlockSpec((r, HID), lambda i: (i, 0)),
          pl.BlockSpec((HEADS, r, D), lambda i: (0, i, 0)),
          pl.BlockSpec((HEADS, r, D), lambda i: (0, i, 0)),
          pl.BlockSpec((HEADS, r, D), lambda i: (0, i, 0)),
      ],
      out_shape=[
          jax.ShapeDtypeStruct((n, HID), f32),
          jax.ShapeDtypeStruct((HEADS, n, D), f32),
          jax.ShapeDtypeStruct((HEADS, n, D), f32),
          jax.ShapeDtypeStruct((HEADS, n, D), f32),
      ],
  )(x, wp, bp, wq, bq, wk, bk, wv, bv)


def _front2(h, wq, bq, wk, bk, wv, bv):
  n = h.shape[0]
  r = _TC_R
  f32 = jnp.float32
  return pl.pallas_call(
      _front2_body,
      grid=(n // r,),
      in_specs=[
          pl.BlockSpec((r, HID), lambda i: (i, 0)),
          _w_spec((HID, HID)), _w_spec((1, HID)),
          _w_spec((HID, HID)), _w_spec((1, HID)),
          _w_spec((HID, HID)), _w_spec((1, HID)),
      ],
      out_specs=[
          pl.BlockSpec((HEADS, r, D), lambda i: (0, i, 0)),
          pl.BlockSpec((HEADS, r, D), lambda i: (0, i, 0)),
          pl.BlockSpec((HEADS, r, D), lambda i: (0, i, 0)),
      ],
      out_shape=[
          jax.ShapeDtypeStruct((HEADS, n, D), f32),
          jax.ShapeDtypeStruct((HEADS, n, D), f32),
          jax.ShapeDtypeStruct((HEADS, n, D), f32),
      ],
  )(h, wq, bq, wk, bk, wv, bv)


def _back(nd, h, wo, bo, skip, relu):
  n = h.shape[0]
  r = _TC_R
  return pl.pallas_call(
      functools.partial(_back_body, relu=relu),
      grid=(n // r,),
      in_specs=[
          pl.BlockSpec((HEADS, r, 24), lambda i: (0, i, 0)),
          pl.BlockSpec((r, HID), lambda i: (i, 0)),
          _w_spec((HID, HID)), _w_spec((1, HID)),
          _w_spec((1, 1)),
      ],
      out_specs=pl.BlockSpec((r, HID), lambda i: (i, 0)),
      out_shape=jax.ShapeDtypeStruct((n, HID), jnp.float32),
  )(nd, h, wo, bo, skip)


_C = 400
_ZR = 136


@functools.cache
def _make_sc_et(n, e):
  assert e % (_C * 16) == 0
  m = e // (_C * 16)
  assert m % 2 == 1 and m >= 3
  mp = (m - 1) // 2
  groups = _C // 16
  rpt = ((n // 16 + 7) // 8) * 8
  n_pad = 16 * rpt
  zr = _ZR if rpt % _ZR == 0 else 8
  f32 = jnp.float32
  i32 = jnp.int32
  mesh = plsc.VectorSubcoreMesh(core_axis_name="c", subcore_axis_name="s")

  @functools.partial(
      pl.kernel,
      mesh=mesh,
      out_type=(jax.ShapeDtypeStruct((HEADS, n_pad, 24), f32),
                jax.ShapeDtypeStruct((HEADS, e), f32)),
      compiler_params=pltpu.CompilerParams(
          use_tc_tiling_on_sc=False, needs_layout_passes=False),
      scratch_types=[
          pltpu.VMEM((2, _C), i32),
          pltpu.VMEM((2, _C), i32),
          pltpu.VMEM((2, _C), i32),
          pltpu.VMEM((2, _C, 16), f32),
          pltpu.VMEM((2, _C, 16), f32),
          pltpu.VMEM((2, _C), f32),
          pltpu.VMEM((272,), f32),
          pltpu.VMEM((16,), f32),
          pltpu.VMEM((2, _C, 24), f32),
          pltpu.VMEM((zr, 24), f32),
          pltpu.VMEM((16, 16), f32),
          pltpu.VMEM_SHARED((n_pad, 24), f32),
          pltpu.VMEM_SHARED((16, 16), f32),
          pltpu.SemaphoreType.DMA,
          pltpu.SemaphoreType.DMA,
          pltpu.SemaphoreType.DMA,
          pltpu.SemaphoreType.DMA,
          pltpu.SemaphoreType.DMA,
          pltpu.SemaphoreType.DMA,
          pltpu.SemaphoreType.DMA,
          pltpu.SemaphoreType.DMA,
          pltpu.SemaphoreType.DMA,
      ],
  )
  def sc_et(src_hbm, dst_hbm, ktab, qtab, vtab, nd_out, alpha_out,
            sidx2, didx2, dscat2, gbuf2, qbuf2, abuf2, scr, mbuf, msgbuf2,
            zbuf, tred, acc, tmax, semk0, semk1, semq0, semq1, semw0, semw1,
            semi0, semi1, semz):
    c = lax.axis_index("c")
    s = lax.axis_index("s")
    iota = lax.iota(i32, 16)
    idx_pitch0 = iota * 17
    idx_15 = (iota * 0) + 15
    zero16 = jnp.zeros((16,), f32)
    sidx = (sidx2.at[0], sidx2.at[1])
    didx = (didx2.at[0], didx2.at[1])
    dscat = (dscat2.at[0], dscat2.at[1])
    gbuf = (gbuf2.at[0], gbuf2.at[1])
    qbuf = (qbuf2.at[0], qbuf2.at[1])
    abuf = (abuf2.at[0], abuf2.at[1])
    msgbuf = (msgbuf2.at[0], msgbuf2.at[1])
    semk = (semk0, semk1)
    semq = (semq0, semq1)
    semw = (semw0, semw1)
    semi = (semi0, semi1)

    def chunk_base(j):
      return (s * m + j) * _C

    def fire_idx(j, p):
      base = chunk_base(j)
      pltpu.async_copy(src_hbm.at[pl.ds(base, _C)], sidx[p], semi[p])
      pltpu.async_copy(dst_hbm.at[pl.ds(base, _C)], didx[p], semi[p])

    def wait_idx(j, p):
      base = chunk_base(j)
      pltpu.make_async_copy(src_hbm.at[pl.ds(base, _C)], sidx[p],
                            semi[p]).wait()
      pltpu.make_async_copy(dst_hbm.at[pl.ds(base, _C)], didx[p],
                            semi[p]).wait()

    def zrow(r, _):
      zbuf[r, pl.ds(8, 16)] = zero16
      zbuf[r, pl.ds(0, 16)] = zero16
      return _
    lax.fori_loop(0, zr, zrow, None)

    def zcopy(z, _):
      pltpu.async_copy(zbuf, acc.at[pl.ds(s * rpt + z * zr, zr), :], semz)
      return _
    lax.fori_loop(0, rpt // zr, zcopy, None)

    for p in range(2):
      def zmrow(r, _, p=p):
        msgbuf[p][r, pl.ds(8, 16)] = zero16
        return _
      lax.fori_loop(0, _C, zmrow, None)

    mbuf[...] = jnp.full((16,), -1e30, f32)

    def fire_a(j, p):
      wait_idx(j, p)
      pltpu.async_copy(ktab.at[c].at[sidx[p]], gbuf[p], semk[p])
      pltpu.async_copy(qtab.at[c].at[didx[p]], qbuf[p], semq[p])

    def comp_a(j, p, wait_w, next_j=None, next_guard=None):
      base = chunk_base(j)
      pltpu.make_async_copy(ktab.at[c].at[sidx[p]], gbuf[p], semk[p]).wait()
      pltpu.make_async_copy(qtab.at[c].at[didx[p]], qbuf[p], semq[p]).wait()
      def _wait_w():
        pltpu.make_async_copy(
            abuf[p], alpha_out.at[c, pl.ds(base, _C)], semw[p]).wait()
      if wait_w is True:
        _wait_w()
      elif wait_w is not None:
        pl.when(wait_w)(_wait_w)
      if next_j is not None:
        if next_guard is None:
          fire_idx(next_j, p)
        else:
          pl.when(next_guard)(lambda: fire_idx(next_j, p))

      def group_a(g, _):
        for ee in range(16):
          i = g * 16 + ee
          scr[pl.ds(17 * ee, 16)] = gbuf[p][i] * qbuf[p][i]
        avec = jnp.zeros((16,), f32)
        for d in range(16):
          avec = avec + plsc.load_gather(scr, [idx_pitch0 + d])
        abuf[p][pl.ds(g * 16, 16)] = avec
        mbuf[...] = jnp.maximum(mbuf[...], avec)
        return _
      lax.fori_loop(0, groups, group_a, None)
      pltpu.async_copy(abuf[p], alpha_out.at[c, pl.ds(base, _C)], semw[p])

    fire_idx(0, 0)
    fire_idx(1, 1)
    fire_a(0, 0)
    fire_a(1, 1)

    def body_a(j2, _):
      j0 = 2 * j2
      comp_a(j0, 0, j2 > 0, next_j=j0 + 2)
      fire_a(j0 + 2, 0)
      comp_a(j0 + 1, 1, j2 > 0, next_j=j0 + 3, next_guard=j2 < mp - 1)
      @pl.when(j2 < mp - 1)
      def _():
        fire_a(j0 + 3, 1)
      return _
    lax.fori_loop(0, mp, body_a, None)
    comp_a(m - 1, 0, True)
    pltpu.make_async_copy(
        abuf[0], alpha_out.at[c, pl.ds(chunk_base(m - 1), _C)], semw[0]).wait()
    pltpu.make_async_copy(
        abuf[1], alpha_out.at[c, pl.ds(chunk_base(m - 2), _C)], semw[1]).wait()

    def zdrain(z, _):
      pltpu.make_async_copy(
          zbuf, acc.at[pl.ds(s * rpt + z * zr, zr), :], semz).wait()
      return _
    lax.fori_loop(0, rpt // zr, zdrain, None)

    pltpu.sync_copy(mbuf, tmax.at[s])
    plsc.subcore_barrier()
    pltpu.sync_copy(tmax, tred)
    mx = jnp.full((16,), -1e30, f32)
    for r in range(16):
      mx = jnp.maximum(mx, tred[r])
    scr[pl.ds(0, 16)] = plsc.cummax(mx)
    svec = plsc.load_gather(scr, [idx_15])

    def fire_b(j, p):
      base = chunk_base(j)
      wait_idx(j, p)
      pltpu.async_copy(vtab.at[c].at[sidx[p]], gbuf[p], semk[p])
      pltpu.async_copy(alpha_out.at[c, pl.ds(base, _C)], abuf[p], semq[p])

    def comp_b(j, p, wait_w, next_j=None, next_guard=None):
      base = chunk_base(j)
      pltpu.make_async_copy(vtab.at[c].at[sidx[p]], gbuf[p], semk[p]).wait()
      pltpu.make_async_copy(
          alpha_out.at[c, pl.ds(base, _C)], abuf[p], semq[p]).wait()
      def _wait_w():
        pltpu.make_async_copy(msgbuf[p], acc.at[dscat[p]], semw[p]).wait()
      if wait_w is True:
        _wait_w()
      elif wait_w is not None:
        pl.when(wait_w)(_wait_w)

      def dcopy(i, _):
        dscat[p][pl.ds(i * 16, 16)] = didx[p][pl.ds(i * 16, 16)]
        return _
      lax.fori_loop(0, _C // 16, dcopy, None)
      if next_j is not None:
        if next_guard is None:
          fire_idx(next_j, p)
        else:
          pl.when(next_guard)(lambda: fire_idx(next_j, p))

      def group_b(g, _):
        evec = iota + g * 16
        wvec = jnp.exp(abuf[p][pl.ds(g * 16, 16)] - svec)
        scr[pl.ds(0, 16)] = wvec
        plsc.store_scatter(msgbuf[p], [evec, (iota * 0) + 16], wvec)
        for ee in range(16):
          i = g * 16 + ee
          sp = plsc.load_gather(scr, [(iota * 0) + ee])
          msgbuf[p][i, pl.ds(0, 16)] = sp * gbuf[p][i]
        return _
      lax.fori_loop(0, groups, group_b, None)
      pltpu.async_copy(msgbuf[p], acc.at[dscat[p]], semw[p], add=True)

    fire_idx(0, 0)
    fire_idx(1, 1)
    fire_b(0, 0)
    fire_b(1, 1)

    def body_b(j2, _):
      j0 = 2 * j2
      comp_b(j0, 0, j2 > 0, next_j=j0 + 2)
      fire_b(j0 + 2, 0)
      comp_b(j0 + 1, 1, j2 > 0, next_j=j0 + 3, next_guard=j2 < mp - 1)
      @pl.when(j2 < mp - 1)
      def _():
        fire_b(j0 + 3, 1)
      return _
    lax.fori_loop(0, mp, body_b, None)
    comp_b(m - 1, 0, True)
    pltpu.make_async_copy(msgbuf[0], acc.at[dscat[0]], semw[0]).wait()
    pltpu.make_async_copy(msgbuf[1], acc.at[dscat[1]], semw[1]).wait()

    plsc.subcore_barrier()
    pltpu.sync_copy(acc.at[pl.ds(s * rpt, rpt), :],
                    nd_out.at[c, pl.ds(s * rpt, rpt), :])

  return sc_et


def _eff_weights(cp, nt, rel):
  scale = 1.0 / math.sqrt(D)
  a = cp["a_" + rel]
  m = cp["m_" + rel]
  p = cp["p_" + rel]
  za = jnp.zeros((HID, HID), jnp.float32)
  za = za.at[:D, :D].set(a[0] * (p[0] * scale))
  za = za.at[D:, D:].set(a[1] * (p[1] * scale))
  zm = jnp.zeros((HID, HID), jnp.float32)
  zm = zm.at[:D, :D].set(m[0])
  zm = zm.at[D:, D:].set(m[1])
  wk = cp["Wk_" + nt] @ za
  bk = (cp["bk_" + nt] @ za).reshape(1, HID)
  wv = cp["Wv_" + nt] @ zm
  bv = (cp["bv_" + nt] @ zm).reshape(1, HID)
  wq = cp["Wq_" + nt]
  bq = cp["bq_" + nt].reshape(1, HID)
  return wq, bq, wk, bk, wv, bv


def kernel(x_user, x_item, edge_index_user_item, edge_index_item_user, params):
  n = x_user.shape[0]
  e = edge_index_user_item.shape[1]
  sc_et = _make_sc_et(n, e)

  src_ui = edge_index_user_item[0]
  dst_ui = edge_index_user_item[1]
  src_iu = edge_index_item_user[0]
  dst_iu = edge_index_item_user[1]

  c1, c2 = params["c1"], params["c2"]
  eff1_u = _eff_weights(c1, "user", "to")
  eff1_i = _eff_weights(c1, "item", "rev")
  eff2_u = _eff_weights(c2, "user", "to")
  eff2_i = _eff_weights(c2, "item", "rev")

  h_u, qt_u, kt_u, vt_u = _front1(
      x_user, params["Wp_user"], params["bp_user"].reshape(1, HID), *eff1_u)
  h_i, qt_i, kt_i, vt_i = _front1(
      x_item, params["Wp_item"], params["bp_item"].reshape(1, HID), *eff1_i)

  nd_to, _ = sc_et(src_ui, dst_ui, kt_u, qt_i, vt_u)
  nd_rev, _ = sc_et(src_iu, dst_iu, kt_i, qt_u, vt_i)

  h2_u = _back(nd_rev, h_u, c1["Wo_user"], c1["bo_user"].reshape(1, HID),
               c1["skip_user"].reshape(1, 1), relu=True)
  h2_i = _back(nd_to, h_i, c1["Wo_item"], c1["bo_item"].reshape(1, HID),
               c1["skip_item"].reshape(1, 1), relu=True)

  qt2_u, kt2_u, vt2_u = _front2(h2_u, *eff2_u)
  qt2_i, kt2_i, vt2_i = _front2(h2_i, *eff2_i)

  nd_to2, _ = sc_et(src_ui, dst_ui, kt2_u, qt2_i, vt2_u)
  nd_rev2, _ = sc_et(src_iu, dst_iu, kt2_i, qt2_u, vt2_i)

  out_u = _back(nd_rev2, h2_u, c2["Wo_user"], c2["bo_user"].reshape(1, HID),
                c2["skip_user"].reshape(1, 1), relu=False)
  out_i = _back(nd_to2, h2_i, c2["Wo_item"], c2["bo_item"].reshape(1, HID),
                c2["skip_item"].reshape(1, 1), relu=False)
  return out_u, out_i
